# Initial kernel scaffold; baseline (speedup 1.0000x reference)
import jax
import jax.numpy as jnp
from jax.experimental import pallas as pl


def kernel(eplus_obs_vec, feature_indices, edge_index, W_enc, b_enc, W_gcn, b_gcn, W_q1, b_q1, W_q2, b_q2):
    raise NotImplementedError("write your pallas kernel here")


# speedup vs baseline: 27.9103x; 27.9103x over previous
import jax
import jax.numpy as jnp
from jax import lax
from jax.experimental import pallas as pl
from jax.experimental.pallas import tpu as pltpu
from jax.experimental.pallas import tpu_sc as plsc

N_NODES = 10000
NUM_FEATURES = 16
OBS_LEN = N_NODES * NUM_FEATURES
N_EDGES = 320000
H = 128
ACTION_DIM = 4

NC = 2
NS = 16
NW = NC * NS

B = 128
FB = 40
FI_PAD = NW * FB * B
EB = 80
E_PAD = NW * EB * B
N_ACC = 10240
RPS = N_ACC // NS

_f32 = jnp.float32


def _sc_a_body(obs_hbm, fi_hbm, dst_hbm, zeros1_hbm, ones_hbm,
               feat_out, deg_out,
               fi_v, g_v, dst_v, ones_v, deg_sh, sem):
    c = lax.axis_index("c")
    s = lax.axis_index("s")
    w = s * NC + c
    pltpu.sync_copy(zeros1_hbm, deg_sh.at[pl.ds(s * RPS, RPS)])
    pltpu.sync_copy(ones_hbm, ones_v)
    pltpu.sync_copy(fi_hbm.at[pl.ds(w * FB, FB)], fi_v)
    copies = [pltpu.async_copy(obs_hbm.at[fi_v.at[j]], g_v.at[j], sem)
              for j in range(FB)]
    for cp in copies:
        cp.wait()
    pltpu.sync_copy(g_v, feat_out.at[pl.ds(w * FB, FB)])
    pltpu.sync_copy(dst_hbm.at[pl.ds(w * EB, EB)], dst_v)
    plsc.subcore_barrier()

    def deg_body(j, carry):
        pltpu.sync_copy(ones_v, deg_sh.at[dst_v.at[j]], add=True)
        return carry

    lax.fori_loop(0, EB, deg_body, 0)
    plsc.subcore_barrier()
    pltpu.sync_copy(deg_sh.at[pl.ds(s * RPS, RPS)],
                    deg_out.at[c, 0, pl.ds(s * RPS, RPS)])


_sc_a = pl.kernel(
    _sc_a_body,
    out_type=(jax.ShapeDtypeStruct((NW * FB, B), _f32),
              jax.ShapeDtypeStruct((NC, 1, N_ACC), _f32)),
    mesh=plsc.VectorSubcoreMesh(core_axis_name="c", subcore_axis_name="s",
                                num_cores=NC, num_subcores=NS),
    scratch_types=(
        pltpu.VMEM((FB, B), jnp.int32),
        pltpu.VMEM((FB, B), _f32),
        pltpu.VMEM((EB, B), jnp.int32),
        pltpu.VMEM((B,), _f32),
        pltpu.VMEM_SHARED((N_ACC,), _f32),
        pltpu.SemaphoreType.DMA,
    ),
)


def _sc_b_body(y_hbm, src_hbm, dst_hbm, zeros2_hbm,
               agg_out,
               src_v, dst_v, msg_v, agg_sh, sem):
    c = lax.axis_index("c")
    s = lax.axis_index("s")
    w = s * NC + c
    pltpu.sync_copy(src_hbm.at[pl.ds(w * EB, EB)], src_v)
    pltpu.sync_copy(dst_hbm.at[pl.ds(w * EB, EB)], dst_v)
    for t in range(RPS // B):
        pltpu.sync_copy(zeros2_hbm, agg_sh.at[pl.ds(s * RPS + t * B, B)])
    plsc.subcore_barrier()

    def body(j, carry):
        pltpu.async_copy(y_hbm.at[src_v.at[j]], msg_v, sem).wait()
        pltpu.sync_copy(msg_v, agg_sh.at[dst_v.at[j]], add=True)
        return carry

    lax.fori_loop(0, EB, body, 0)
    plsc.subcore_barrier()
    pltpu.sync_copy(agg_sh.at[pl.ds(s * RPS, RPS)],
                    agg_out.at[c, pl.ds(s * RPS, RPS)])


_sc_b = pl.kernel(
    _sc_b_body,
    out_type=jax.ShapeDtypeStruct((NC, N_ACC, H), _f32),
    mesh=plsc.VectorSubcoreMesh(core_axis_name="c", subcore_axis_name="s",
                                num_cores=NC, num_subcores=NS),
    scratch_types=(
        pltpu.VMEM((EB, B), jnp.int32),
        pltpu.VMEM((EB, B), jnp.int32),
        pltpu.VMEM((B, H), _f32),
        pltpu.VMEM_SHARED((N_ACC, H), _f32),
        pltpu.SemaphoreType.DMA,
    ),
)


_RB = 1000


def _tc1_body(feat_ref, deg_ref, we_ref, be_ref, wg_ref, y_ref, dinv_ref):
    f = feat_ref[...]
    h1 = jnp.maximum(
        jnp.dot(f, we_ref[...], preferred_element_type=_f32) + be_ref[...], 0.0)
    xw = jnp.dot(h1, wg_ref[...], preferred_element_type=_f32)
    deg = deg_ref[0, 0, 0, :] + deg_ref[1, 0, 0, :] + 1.0
    dinv = lax.rsqrt(deg)
    y_ref[...] = xw * dinv[:, None]
    dinv_ref[...] = dinv[None, None, :]


def _tc2_body(agg_ref, y_ref, dinv_ref, bg_ref, wq1_ref, bq1_ref,
              wq2_ref, bq2_ref, q_ref):
    pre = (agg_ref[0] + agg_ref[1] + y_ref[...]) * dinv_ref[0, 0, :][:, None]
    h2 = jnp.maximum(pre + bg_ref[...], 0.0)
    t = jnp.maximum(
        jnp.dot(h2, wq1_ref[...], preferred_element_type=_f32) + bq1_ref[...],
        0.0)
    q_ref[...] = jnp.dot(t, wq2_ref[...], preferred_element_type=_f32) \
        + bq2_ref[...]


def kernel(eplus_obs_vec, feature_indices, edge_index, W_enc, b_enc,
           W_gcn, b_gcn, W_q1, b_q1, W_q2, b_q2):
    fi = feature_indices.reshape(-1)
    pad_fi = (jnp.arange(FI_PAD - OBS_LEN, dtype=jnp.int32) * 41) % OBS_LEN
    fi_p = jnp.concatenate([fi, pad_fi]).reshape(NW * FB, B)

    src = edge_index[0]
    dst = edge_index[1]
    npad = E_PAD - N_EDGES
    pad_src = (jnp.arange(npad, dtype=jnp.int32) * 37) % N_NODES
    pad_dst = N_NODES + (jnp.arange(npad, dtype=jnp.int32) % (N_ACC - N_NODES))
    src_p = jnp.concatenate([src, pad_src]).reshape(NW * EB, B)
    dst_p = jnp.concatenate([dst, pad_dst]).reshape(NW * EB, B)

    zeros1 = jnp.zeros((RPS,), _f32)
    zeros2 = jnp.zeros((B, H), _f32)
    ones1 = jnp.ones((B,), _f32)

    feat_flat, deg_parts = _sc_a(eplus_obs_vec, fi_p, dst_p, zeros1, ones1)
    feature = feat_flat.reshape(-1)[:OBS_LEN].reshape(N_NODES, NUM_FEATURES)
    deg3 = deg_parts[:, 0, :N_NODES].reshape(NC, N_NODES // _RB, 1, _RB)

    y, dinv = pl.pallas_call(
        _tc1_body,
        grid=(N_NODES // _RB,),
        in_specs=[
            pl.BlockSpec((_RB, NUM_FEATURES), lambda i: (i, 0)),
            pl.BlockSpec((NC, 1, 1, _RB), lambda i: (0, i, 0, 0)),
            pl.BlockSpec((NUM_FEATURES, H), lambda i: (0, 0)),
            pl.BlockSpec((1, H), lambda i: (0, 0)),
            pl.BlockSpec((H, H), lambda i: (0, 0)),
        ],
        out_specs=[
            pl.BlockSpec((_RB, H), lambda i: (i, 0)),
            pl.BlockSpec((1, 1, _RB), lambda i: (i, 0, 0)),
        ],
        out_shape=[
            jax.ShapeDtypeStruct((N_NODES, H), _f32),
            jax.ShapeDtypeStruct((N_NODES // _RB, 1, _RB), _f32),
        ],
    )(feature, deg3, W_enc, b_enc.reshape(1, H), W_gcn)

    agg = _sc_b(y, src_p, dst_p, zeros2)

    wq2p = jnp.zeros((H, H), _f32).at[:, :ACTION_DIM].set(W_q2)
    bq2p = jnp.zeros((1, H), _f32).at[0, :ACTION_DIM].set(b_q2)
    q = pl.pallas_call(
        _tc2_body,
        grid=(N_NODES // _RB,),
        in_specs=[
            pl.BlockSpec((NC, _RB, H), lambda i: (0, i, 0)),
            pl.BlockSpec((_RB, H), lambda i: (i, 0)),
            pl.BlockSpec((1, 1, _RB), lambda i: (i, 0, 0)),
            pl.BlockSpec((1, H), lambda i: (0, 0)),
            pl.BlockSpec((H, H), lambda i: (0, 0)),
            pl.BlockSpec((1, H), lambda i: (0, 0)),
            pl.BlockSpec((H, H), lambda i: (0, 0)),
            pl.BlockSpec((1, H), lambda i: (0, 0)),
        ],
        out_specs=pl.BlockSpec((_RB, H), lambda i: (i, 0)),
        out_shape=jax.ShapeDtypeStruct((N_NODES, H), _f32),
    )(agg, y, dinv, b_gcn.reshape(1, H), W_q1, b_q1.reshape(1, H),
      wq2p, bq2p)
    return q[:, :ACTION_DIM]


# speedup vs baseline: 32.9138x; 1.1793x over previous
import jax
import jax.numpy as jnp
from jax import lax
from jax.experimental import pallas as pl
from jax.experimental.pallas import tpu as pltpu
from jax.experimental.pallas import tpu_sc as plsc

N_NODES = 10000
NUM_FEATURES = 16
OBS_LEN = N_NODES * NUM_FEATURES
N_EDGES = 320000
H = 128
ACTION_DIM = 4

NC = 2
NS = 16
NW = NC * NS

B = 128
FB = 40
FI_PAD = NW * FB * B
EB = 80
E_PAD = NW * EB * B
N_ACC = 10240
RPS = N_ACC // NS

_f32 = jnp.float32


def _sc_a_body(obs_hbm, fi_hbm, dst_hbm, zeros1_hbm, ones_hbm,
               feat_out, deg_out,
               fi_v, g_v, dst_v, ones_v, deg_sh, sem):
    c = lax.axis_index("c")
    s = lax.axis_index("s")
    w = s * NC + c
    pltpu.sync_copy(zeros1_hbm, deg_sh.at[pl.ds(s * RPS, RPS)])
    pltpu.sync_copy(ones_hbm, ones_v)
    pltpu.sync_copy(fi_hbm.at[pl.ds(w * FB, FB)], fi_v)
    copies = [pltpu.async_copy(obs_hbm.at[fi_v.at[j]], g_v.at[j], sem)
              for j in range(FB)]
    for cp in copies:
        cp.wait()
    pltpu.sync_copy(g_v, feat_out.at[pl.ds(w * FB, FB)])
    pltpu.sync_copy(dst_hbm.at[pl.ds(w * EB, EB)], dst_v)
    plsc.subcore_barrier()

    def deg_body(j, carry):
        pltpu.sync_copy(ones_v, deg_sh.at[dst_v.at[j]], add=True)
        return carry

    lax.fori_loop(0, EB, deg_body, 0)
    plsc.subcore_barrier()
    pltpu.sync_copy(deg_sh.at[pl.ds(s * RPS, RPS)],
                    deg_out.at[c, 0, pl.ds(s * RPS, RPS)])


_sc_a = pl.kernel(
    _sc_a_body,
    out_type=(jax.ShapeDtypeStruct((NW * FB, B), _f32),
              jax.ShapeDtypeStruct((NC, 1, N_ACC), _f32)),
    mesh=plsc.VectorSubcoreMesh(core_axis_name="c", subcore_axis_name="s",
                                num_cores=NC, num_subcores=NS),
    scratch_types=(
        pltpu.VMEM((FB, B), jnp.int32),
        pltpu.VMEM((FB, B), _f32),
        pltpu.VMEM((EB, B), jnp.int32),
        pltpu.VMEM((B,), _f32),
        pltpu.VMEM_SHARED((N_ACC,), _f32),
        pltpu.SemaphoreType.DMA,
    ),
)


_G = 16
_NG = EB // _G


def _sc_b_body(y_hbm, src_hbm, dst_hbm, zeros2_hbm,
               agg_out,
               src_v, dst_v, msg_v, agg_sh, gsem, ssem, isem):
    c = lax.axis_index("c")
    s = lax.axis_index("s")
    w = s * NC + c
    for t in range(RPS // B):
        pltpu.sync_copy(zeros2_hbm, agg_sh.at[pl.ds(s * RPS + t * B, B)])

    def load_idx(g):
        base = w * EB + g * _G
        return (pltpu.async_copy(src_hbm.at[pl.ds(base, _G)],
                                 src_v.at[g % 2], isem),
                pltpu.async_copy(dst_hbm.at[pl.ds(base, _G)],
                                 dst_v.at[g % 2], isem))

    def gather(j):
        gidx = (j // _G) % 2
        return pltpu.async_copy(y_hbm.at[src_v.at[gidx].at[j % _G]],
                                msg_v.at[j % 2], gsem)

    def scatter(j):
        gidx = (j // _G) % 2
        return pltpu.async_copy(msg_v.at[j % 2],
                                agg_sh.at[dst_v.at[gidx].at[j % _G]],
                                ssem, add=True)

    il = [None] * (_NG + 1)
    il[0] = load_idx(0)
    for d in il[0]:
        d.wait()
    plsc.subcore_barrier()

    gd = [None] * EB
    sd = [None] * EB
    gd[0] = gather(0)
    for j in range(EB):
        gd[j].wait()
        sd[j] = scatter(j)
        if j >= 1:
            sd[j - 1].wait()
        if j % _G == 0 and (g := j // _G) + 1 < _NG:
            il[g + 1] = load_idx(g + 1)
        nxt = j + 1
        if nxt < EB:
            if nxt % _G == 0:
                for d in il[nxt // _G]:
                    d.wait()
            gd[nxt] = gather(nxt)
    sd[EB - 1].wait()
    plsc.subcore_barrier()
    pltpu.sync_copy(agg_sh.at[pl.ds(s * RPS, RPS)],
                    agg_out.at[c, pl.ds(s * RPS, RPS)])


_sc_b = pl.kernel(
    _sc_b_body,
    out_type=jax.ShapeDtypeStruct((NC, N_ACC, H), _f32),
    mesh=plsc.VectorSubcoreMesh(core_axis_name="c", subcore_axis_name="s",
                                num_cores=NC, num_subcores=NS),
    scratch_types=(
        pltpu.VMEM((2, _G, B), jnp.int32),
        pltpu.VMEM((2, _G, B), jnp.int32),
        pltpu.VMEM((2, B, H), _f32),
        pltpu.VMEM_SHARED((N_ACC, H), _f32),
        pltpu.SemaphoreType.DMA,
        pltpu.SemaphoreType.DMA,
        pltpu.SemaphoreType.DMA,
    ),
)


_RB = 1000


def _tc1_body(feat_ref, deg_ref, we_ref, be_ref, wg_ref, y_ref, dinv_ref):
    f = feat_ref[...]
    h1 = jnp.maximum(
        jnp.dot(f, we_ref[...], preferred_element_type=_f32) + be_ref[...], 0.0)
    xw = jnp.dot(h1, wg_ref[...], preferred_element_type=_f32)
    deg = deg_ref[0, 0, 0, :] + deg_ref[1, 0, 0, :] + 1.0
    dinv = lax.rsqrt(deg)
    y_ref[...] = xw * dinv[:, None]
    dinv_ref[...] = dinv[None, None, :]


def _tc2_body(agg_ref, y_ref, dinv_ref, bg_ref, wq1_ref, bq1_ref,
              wq2_ref, bq2_ref, q_ref):
    pre = (agg_ref[0] + agg_ref[1] + y_ref[...]) * dinv_ref[0, 0, :][:, None]
    h2 = jnp.maximum(pre + bg_ref[...], 0.0)
    t = jnp.maximum(
        jnp.dot(h2, wq1_ref[...], preferred_element_type=_f32) + bq1_ref[...],
        0.0)
    q_ref[...] = jnp.dot(t, wq2_ref[...], preferred_element_type=_f32) \
        + bq2_ref[...]


def kernel(eplus_obs_vec, feature_indices, edge_index, W_enc, b_enc,
           W_gcn, b_gcn, W_q1, b_q1, W_q2, b_q2):
    fi = feature_indices.reshape(-1)
    pad_fi = (jnp.arange(FI_PAD - OBS_LEN, dtype=jnp.int32) * 41) % OBS_LEN
    fi_p = jnp.concatenate([fi, pad_fi]).reshape(NW * FB, B)

    src = edge_index[0]
    dst = edge_index[1]
    npad = E_PAD - N_EDGES
    pad_src = (jnp.arange(npad, dtype=jnp.int32) * 37) % N_NODES
    pad_dst = N_NODES + (jnp.arange(npad, dtype=jnp.int32) % (N_ACC - N_NODES))
    src_p = jnp.concatenate([src, pad_src]).reshape(NW * EB, B)
    dst_p = jnp.concatenate([dst, pad_dst]).reshape(NW * EB, B)

    zeros1 = jnp.zeros((RPS,), _f32)
    zeros2 = jnp.zeros((B, H), _f32)
    ones1 = jnp.ones((B,), _f32)

    feat_flat, deg_parts = _sc_a(eplus_obs_vec, fi_p, dst_p, zeros1, ones1)
    feature = feat_flat.reshape(-1)[:OBS_LEN].reshape(N_NODES, NUM_FEATURES)
    deg3 = deg_parts[:, 0, :N_NODES].reshape(NC, N_NODES // _RB, 1, _RB)

    y, dinv = pl.pallas_call(
        _tc1_body,
        grid=(N_NODES // _RB,),
        in_specs=[
            pl.BlockSpec((_RB, NUM_FEATURES), lambda i: (i, 0)),
            pl.BlockSpec((NC, 1, 1, _RB), lambda i: (0, i, 0, 0)),
            pl.BlockSpec((NUM_FEATURES, H), lambda i: (0, 0)),
            pl.BlockSpec((1, H), lambda i: (0, 0)),
            pl.BlockSpec((H, H), lambda i: (0, 0)),
        ],
        out_specs=[
            pl.BlockSpec((_RB, H), lambda i: (i, 0)),
            pl.BlockSpec((1, 1, _RB), lambda i: (i, 0, 0)),
        ],
        out_shape=[
            jax.ShapeDtypeStruct((N_NODES, H), _f32),
            jax.ShapeDtypeStruct((N_NODES // _RB, 1, _RB), _f32),
        ],
    )(feature, deg3, W_enc, b_enc.reshape(1, H), W_gcn)

    agg = _sc_b(y, src_p, dst_p, zeros2)

    wq2p = jnp.zeros((H, 8), _f32).at[:, :ACTION_DIM].set(W_q2)
    bq2p = jnp.zeros((1, 8), _f32).at[0, :ACTION_DIM].set(b_q2)
    q = pl.pallas_call(
        _tc2_body,
        grid=(N_NODES // _RB,),
        in_specs=[
            pl.BlockSpec((NC, _RB, H), lambda i: (0, i, 0)),
            pl.BlockSpec((_RB, H), lambda i: (i, 0)),
            pl.BlockSpec((1, 1, _RB), lambda i: (i, 0, 0)),
            pl.BlockSpec((1, H), lambda i: (0, 0)),
            pl.BlockSpec((H, H), lambda i: (0, 0)),
            pl.BlockSpec((1, H), lambda i: (0, 0)),
            pl.BlockSpec((H, 8), lambda i: (0, 0)),
            pl.BlockSpec((1, 8), lambda i: (0, 0)),
        ],
        out_specs=pl.BlockSpec((_RB, 8), lambda i: (i, 0)),
        out_shape=jax.ShapeDtypeStruct((N_NODES, 8), _f32),
    )(agg, y, dinv, b_gcn.reshape(1, H), W_q1, b_q1.reshape(1, H),
      wq2p, bq2p)
    return q[:, :ACTION_DIM]


# speedup vs baseline: 36.7671x; 1.1171x over previous
import jax
import jax.numpy as jnp
from jax import lax
from jax.experimental import pallas as pl
from jax.experimental.pallas import tpu as pltpu
from jax.experimental.pallas import tpu_sc as plsc

N_NODES = 10000
NUM_FEATURES = 16
OBS_LEN = N_NODES * NUM_FEATURES
N_EDGES = 320000
H = 128
ACTION_DIM = 4

NC = 2
NS = 16
NW = NC * NS

B = 128
FB = 40
FI_PAD = NW * FB * B
EB = 80
E_PAD = NW * EB * B
N_ACC = 10240
RPS = N_ACC // NS

_f32 = jnp.float32


def _sc_a_body(obs_hbm, fi_hbm, dst_hbm, zeros1_hbm, ones_hbm,
               feat_out, deg_out,
               fi_v, g_v, dst_v, ones_v, deg_sh, sem):
    c = lax.axis_index("c")
    s = lax.axis_index("s")
    w = s * NC + c
    pltpu.sync_copy(zeros1_hbm, deg_sh.at[pl.ds(s * RPS, RPS)])
    pltpu.sync_copy(ones_hbm, ones_v)
    pltpu.sync_copy(fi_hbm.at[pl.ds(w * FB, FB)], fi_v)
    copies = [pltpu.async_copy(obs_hbm.at[fi_v.at[j]], g_v.at[j], sem)
              for j in range(FB)]
    for cp in copies:
        cp.wait()
    pltpu.sync_copy(g_v, feat_out.at[pl.ds(w * FB, FB)])
    pltpu.sync_copy(dst_hbm.at[pl.ds(w * EB, EB)], dst_v)
    plsc.subcore_barrier()

    def deg_body(j, carry):
        pltpu.sync_copy(ones_v, deg_sh.at[dst_v.at[j]], add=True)
        return carry

    lax.fori_loop(0, EB, deg_body, 0)
    plsc.subcore_barrier()
    pltpu.sync_copy(deg_sh.at[pl.ds(s * RPS, RPS)],
                    deg_out.at[c, 0, pl.ds(s * RPS, RPS)])


_sc_a = pl.kernel(
    _sc_a_body,
    out_type=(jax.ShapeDtypeStruct((NW * FB, B), _f32),
              jax.ShapeDtypeStruct((NC, 1, N_ACC), _f32)),
    mesh=plsc.VectorSubcoreMesh(core_axis_name="c", subcore_axis_name="s",
                                num_cores=NC, num_subcores=NS),
    scratch_types=(
        pltpu.VMEM((FB, B), jnp.int32),
        pltpu.VMEM((FB, B), _f32),
        pltpu.VMEM((EB, B), jnp.int32),
        pltpu.VMEM((B,), _f32),
        pltpu.VMEM_SHARED((N_ACC,), _f32),
        pltpu.SemaphoreType.DMA,
    ),
)


_G = 16
_NG = EB // _G


def _sc_b_body(y_hbm, src_hbm, dst_hbm, zeros2_hbm,
               agg_out,
               src_v, dst_v, msg_v, agg_sh, gsem, ssem, isem):
    c = lax.axis_index("c")
    s = lax.axis_index("s")
    w = s * NC + c
    for t in range(RPS // B):
        pltpu.sync_copy(zeros2_hbm, agg_sh.at[pl.ds(s * RPS + t * B, B)])

    def load_idx(g):
        base = w * EB + g * _G
        return (pltpu.async_copy(src_hbm.at[pl.ds(base, _G)],
                                 src_v.at[g % 2], isem),
                pltpu.async_copy(dst_hbm.at[pl.ds(base, _G)],
                                 dst_v.at[g % 2], isem))

    def gather(j):
        gidx = (j // _G) % 2
        return [pltpu.async_copy(
                    y_hbm.at[src_v.at[gidx, j % _G, pl.ds(h * 64, 64)]],
                    msg_v.at[j % 2, pl.ds(h * 64, 64)], gsem)
                for h in range(2)]

    def scatter(j):
        gidx = (j // _G) % 2
        return pltpu.async_copy(msg_v.at[j % 2],
                                agg_sh.at[dst_v.at[gidx].at[j % _G]],
                                ssem, add=True)

    il = [None] * (_NG + 1)
    il[0] = load_idx(0)
    for d in il[0]:
        d.wait()
    plsc.subcore_barrier()

    gd = [None] * EB
    sd = [None] * EB
    gd[0] = gather(0)
    for j in range(EB):
        if j >= 1:
            sd[j - 1].wait()
        if j % _G == 0 and (g := j // _G) + 1 < _NG:
            il[g + 1] = load_idx(g + 1)
        nxt = j + 1
        if nxt < EB:
            if nxt % _G == 0:
                for d in il[nxt // _G]:
                    d.wait()
            gd[nxt] = gather(nxt)
        for d in gd[j]:
            d.wait()
        sd[j] = scatter(j)
    sd[EB - 1].wait()
    plsc.subcore_barrier()
    pltpu.sync_copy(agg_sh.at[pl.ds(s * RPS, RPS)],
                    agg_out.at[c, pl.ds(s * RPS, RPS)])


_sc_b = pl.kernel(
    _sc_b_body,
    out_type=jax.ShapeDtypeStruct((NC, N_ACC, H), _f32),
    mesh=plsc.VectorSubcoreMesh(core_axis_name="c", subcore_axis_name="s",
                                num_cores=NC, num_subcores=NS),
    scratch_types=(
        pltpu.VMEM((2, _G, B), jnp.int32),
        pltpu.VMEM((2, _G, B), jnp.int32),
        pltpu.VMEM((2, B, H), _f32),
        pltpu.VMEM_SHARED((N_ACC, H), _f32),
        pltpu.SemaphoreType.DMA,
        pltpu.SemaphoreType.DMA,
        pltpu.SemaphoreType.DMA,
    ),
)


_RB = 1000


def _tc1_body(feat_ref, deg_ref, we_ref, be_ref, wg_ref, y_ref, dinv_ref):
    f = feat_ref[...]
    h1 = jnp.maximum(
        jnp.dot(f, we_ref[...], preferred_element_type=_f32) + be_ref[...], 0.0)
    xw = jnp.dot(h1, wg_ref[...], preferred_element_type=_f32)
    deg = deg_ref[0, 0, 0, :] + deg_ref[1, 0, 0, :] + 1.0
    dinv = lax.rsqrt(deg)
    y_ref[...] = xw * dinv[:, None]
    dinv_ref[...] = dinv[None, None, :]


def _tc2_body(agg_ref, y_ref, dinv_ref, bg_ref, wq1_ref, bq1_ref,
              wq2_ref, bq2_ref, q_ref):
    pre = (agg_ref[0] + agg_ref[1] + y_ref[...]) * dinv_ref[0, 0, :][:, None]
    h2 = jnp.maximum(pre + bg_ref[...], 0.0)
    t = jnp.maximum(
        jnp.dot(h2, wq1_ref[...], preferred_element_type=_f32) + bq1_ref[...],
        0.0)
    q_ref[...] = jnp.dot(t, wq2_ref[...], preferred_element_type=_f32) \
        + bq2_ref[...]


def kernel(eplus_obs_vec, feature_indices, edge_index, W_enc, b_enc,
           W_gcn, b_gcn, W_q1, b_q1, W_q2, b_q2):
    fi = feature_indices.reshape(-1)
    pad_fi = (jnp.arange(FI_PAD - OBS_LEN, dtype=jnp.int32) * 41) % OBS_LEN
    fi_p = jnp.concatenate([fi, pad_fi]).reshape(NW * FB, B)

    src = edge_index[0]
    dst = edge_index[1]
    npad = E_PAD - N_EDGES
    pad_src = (jnp.arange(npad, dtype=jnp.int32) * 37) % N_NODES
    pad_dst = N_NODES + (jnp.arange(npad, dtype=jnp.int32) % (N_ACC - N_NODES))
    src_p = jnp.concatenate([src, pad_src]).reshape(NW * EB, B)
    dst_p = jnp.concatenate([dst, pad_dst]).reshape(NW * EB, B)

    zeros1 = jnp.zeros((RPS,), _f32)
    zeros2 = jnp.zeros((B, H), _f32)
    ones1 = jnp.ones((B,), _f32)

    feat_flat, deg_parts = _sc_a(eplus_obs_vec, fi_p, dst_p, zeros1, ones1)
    feature = feat_flat.reshape(-1)[:OBS_LEN].reshape(N_NODES, NUM_FEATURES)
    deg3 = deg_parts[:, 0, :N_NODES].reshape(NC, N_NODES // _RB, 1, _RB)

    y, dinv = pl.pallas_call(
        _tc1_body,
        grid=(N_NODES // _RB,),
        in_specs=[
            pl.BlockSpec((_RB, NUM_FEATURES), lambda i: (i, 0)),
            pl.BlockSpec((NC, 1, 1, _RB), lambda i: (0, i, 0, 0)),
            pl.BlockSpec((NUM_FEATURES, H), lambda i: (0, 0)),
            pl.BlockSpec((1, H), lambda i: (0, 0)),
            pl.BlockSpec((H, H), lambda i: (0, 0)),
        ],
        out_specs=[
            pl.BlockSpec((_RB, H), lambda i: (i, 0)),
            pl.BlockSpec((1, 1, _RB), lambda i: (i, 0, 0)),
        ],
        out_shape=[
            jax.ShapeDtypeStruct((N_NODES, H), _f32),
            jax.ShapeDtypeStruct((N_NODES // _RB, 1, _RB), _f32),
        ],
    )(feature, deg3, W_enc, b_enc.reshape(1, H), W_gcn)

    agg = _sc_b(y, src_p, dst_p, zeros2)

    wq2p = jnp.zeros((H, 8), _f32).at[:, :ACTION_DIM].set(W_q2)
    bq2p = jnp.zeros((1, 8), _f32).at[0, :ACTION_DIM].set(b_q2)
    q = pl.pallas_call(
        _tc2_body,
        grid=(N_NODES // _RB,),
        in_specs=[
            pl.BlockSpec((NC, _RB, H), lambda i: (0, i, 0)),
            pl.BlockSpec((_RB, H), lambda i: (i, 0)),
            pl.BlockSpec((1, 1, _RB), lambda i: (i, 0, 0)),
            pl.BlockSpec((1, H), lambda i: (0, 0)),
            pl.BlockSpec((H, H), lambda i: (0, 0)),
            pl.BlockSpec((1, H), lambda i: (0, 0)),
            pl.BlockSpec((H, 8), lambda i: (0, 0)),
            pl.BlockSpec((1, 8), lambda i: (0, 0)),
        ],
        out_specs=pl.BlockSpec((_RB, 8), lambda i: (i, 0)),
        out_shape=jax.ShapeDtypeStruct((N_NODES, 8), _f32),
    )(agg, y, dinv, b_gcn.reshape(1, H), W_q1, b_q1.reshape(1, H),
      wq2p, bq2p)
    return q[:, :ACTION_DIM]


# speedup vs baseline: 38.7018x; 1.0526x over previous
import jax
import jax.numpy as jnp
from jax import lax
from jax.experimental import pallas as pl
from jax.experimental.pallas import tpu as pltpu
from jax.experimental.pallas import tpu_sc as plsc

N_NODES = 10000
NUM_FEATURES = 16
OBS_LEN = N_NODES * NUM_FEATURES
N_EDGES = 320000
H = 128
ACTION_DIM = 4

NC = 2
NS = 16
NW = NC * NS

B = 128
EPW = N_EDGES // NW
FPW = OBS_LEN // NW
N_ACC = 10240
RPS = N_ACC // NS

_EBATCH = [(b * B, B) for b in range(EPW // B)] + [(EPW - EPW % B, EPW % B)]
_FBATCH = [(b * B, B) for b in range(FPW // B)] + [(FPW - FPW % B, FPW % B)]
_GSZ = 2048
_GB = _GSZ // B

_f32 = jnp.float32


def _sc_a_body(obs_hbm, fi_hbm, dst_hbm, zeros1_hbm, ones_hbm,
               feat_out, deg_out,
               fi_v, g_v, dst_v, ones_v, deg_sh, gsem, ssem):
    c = lax.axis_index("c")
    s = lax.axis_index("s")
    w = s * NC + c
    pltpu.sync_copy(zeros1_hbm, deg_sh.at[pl.ds(s * RPS, RPS)])
    pltpu.sync_copy(ones_hbm, ones_v)
    pltpu.sync_copy(fi_hbm.at[pl.ds(w * FPW, FPW)], fi_v)
    gds = [pltpu.async_copy(obs_hbm.at[fi_v.at[pl.ds(off, n)]],
                            g_v.at[pl.ds(off, n)], gsem)
           for off, n in _FBATCH]
    pltpu.sync_copy(dst_hbm.at[pl.ds(w * EPW, EPW)], dst_v)
    for d in gds:
        d.wait()
    pltpu.sync_copy(g_v, feat_out.at[pl.ds(w * FPW, FPW)])
    plsc.subcore_barrier()
    sds = [pltpu.async_copy(ones_v.at[pl.ds(0, n)],
                            deg_sh.at[dst_v.at[pl.ds(off, n)]], ssem,
                            add=True)
           for off, n in _EBATCH]
    for d in sds:
        d.wait()
    plsc.subcore_barrier()
    pltpu.sync_copy(deg_sh.at[pl.ds(s * RPS, RPS)],
                    deg_out.at[c, 0, pl.ds(s * RPS, RPS)])


_sc_a = pl.kernel(
    _sc_a_body,
    out_type=(jax.ShapeDtypeStruct((OBS_LEN,), _f32),
              jax.ShapeDtypeStruct((NC, 1, N_ACC), _f32)),
    mesh=plsc.VectorSubcoreMesh(core_axis_name="c", subcore_axis_name="s",
                                num_cores=NC, num_subcores=NS),
    scratch_types=(
        pltpu.VMEM((FPW,), jnp.int32),
        pltpu.VMEM((FPW,), _f32),
        pltpu.VMEM((EPW,), jnp.int32),
        pltpu.VMEM((B,), _f32),
        pltpu.VMEM_SHARED((N_ACC,), _f32),
        pltpu.SemaphoreType.DMA,
        pltpu.SemaphoreType.DMA,
    ),
)


def _sc_b_body(y_hbm, src_hbm, dst_hbm, zeros2_hbm,
               agg_out,
               src_v0, src_v1, dst_v0, dst_v1, msg_v, agg_sh,
               gsem, ssem, isem):
    src_b = (src_v0, src_v1)
    dst_b = (dst_v0, dst_v1)
    c = lax.axis_index("c")
    s = lax.axis_index("s")
    w = s * NC + c
    for t in range(RPS // B):
        pltpu.sync_copy(zeros2_hbm, agg_sh.at[pl.ds(s * RPS + t * B, B)])

    nb = len(_EBATCH)

    def load_idx(g):
        base = w * EPW + g * _GSZ
        gsz = min(_GSZ, EPW - g * _GSZ)
        return (pltpu.async_copy(src_hbm.at[pl.ds(base, gsz)],
                                 src_b[g % 2].at[pl.ds(0, gsz)], isem),
                pltpu.async_copy(dst_hbm.at[pl.ds(base, gsz)],
                                 dst_b[g % 2].at[pl.ds(0, gsz)], isem))

    def gather(j):
        off, n = _EBATCH[j]
        goff = off - (off // _GSZ) * _GSZ
        return pltpu.async_copy(
            y_hbm.at[src_b[(j // _GB) % 2].at[pl.ds(goff, n)]],
            msg_v.at[j % 2, pl.ds(0, n)], gsem)

    def scatter(j):
        off, n = _EBATCH[j]
        goff = off - (off // _GSZ) * _GSZ
        return pltpu.async_copy(
            msg_v.at[j % 2, pl.ds(0, n)],
            agg_sh.at[dst_b[(j // _GB) % 2].at[pl.ds(goff, n)]],
            ssem, add=True)

    ngrp = (nb + _GB - 1) // _GB
    il = [None] * ngrp
    il[0] = load_idx(0)
    for d in il[0]:
        d.wait()
    plsc.subcore_barrier()

    gd = [None] * nb
    sd = [None] * nb
    gd[0] = gather(0)
    for j in range(nb):
        if j >= 1:
            sd[j - 1].wait()
        if j % _GB == 0 and (g := j // _GB) + 1 < ngrp:
            il[g + 1] = load_idx(g + 1)
        nxt = j + 1
        if nxt < nb:
            if nxt % _GB == 0:
                for d in il[nxt // _GB]:
                    d.wait()
            gd[nxt] = gather(nxt)
        gd[j].wait()
        sd[j] = scatter(j)
    sd[nb - 1].wait()
    plsc.subcore_barrier()
    pltpu.sync_copy(agg_sh.at[pl.ds(s * RPS, RPS)],
                    agg_out.at[c, pl.ds(s * RPS, RPS)])


_sc_b = pl.kernel(
    _sc_b_body,
    out_type=jax.ShapeDtypeStruct((NC, N_ACC, H), _f32),
    mesh=plsc.VectorSubcoreMesh(core_axis_name="c", subcore_axis_name="s",
                                num_cores=NC, num_subcores=NS),
    scratch_types=(
        pltpu.VMEM((_GSZ,), jnp.int32),
        pltpu.VMEM((_GSZ,), jnp.int32),
        pltpu.VMEM((_GSZ,), jnp.int32),
        pltpu.VMEM((_GSZ,), jnp.int32),
        pltpu.VMEM((2, B, H), _f32),
        pltpu.VMEM_SHARED((N_ACC, H), _f32),
        pltpu.SemaphoreType.DMA,
        pltpu.SemaphoreType.DMA,
        pltpu.SemaphoreType.DMA,
    ),
)


_RB = 1000


def _tc1_body(feat_ref, deg_ref, we_ref, be_ref, wg_ref, y_ref, dinv_ref):
    f = feat_ref[...]
    h1 = jnp.maximum(
        jnp.dot(f, we_ref[...], preferred_element_type=_f32) + be_ref[...], 0.0)
    xw = jnp.dot(h1, wg_ref[...], preferred_element_type=_f32)
    deg = deg_ref[0, 0, 0, :] + deg_ref[1, 0, 0, :] + 1.0
    dinv = lax.rsqrt(deg)
    y_ref[...] = xw * dinv[:, None]
    dinv_ref[...] = dinv[None, None, :]


def _tc2_body(agg_ref, y_ref, dinv_ref, bg_ref, wq1_ref, bq1_ref,
              wq2_ref, bq2_ref, q_ref):
    pre = (agg_ref[0] + agg_ref[1] + y_ref[...]) * dinv_ref[0, 0, :][:, None]
    h2 = jnp.maximum(pre + bg_ref[...], 0.0)
    t = jnp.maximum(
        jnp.dot(h2, wq1_ref[...], preferred_element_type=_f32) + bq1_ref[...],
        0.0)
    q_ref[...] = jnp.dot(t, wq2_ref[...], preferred_element_type=_f32) \
        + bq2_ref[...]


def kernel(eplus_obs_vec, feature_indices, edge_index, W_enc, b_enc,
           W_gcn, b_gcn, W_q1, b_q1, W_q2, b_q2):
    fi1 = feature_indices.reshape(-1)
    src1 = edge_index[0]
    dst1 = edge_index[1]

    zeros1 = jnp.zeros((RPS,), _f32)
    zeros2 = jnp.zeros((B, H), _f32)
    ones1 = jnp.ones((B,), _f32)

    feat_flat, deg_parts = _sc_a(eplus_obs_vec, fi1, dst1, zeros1, ones1)
    feature = feat_flat.reshape(N_NODES, NUM_FEATURES)
    deg3 = deg_parts[:, 0, :N_NODES].reshape(NC, N_NODES // _RB, 1, _RB)

    y, dinv = pl.pallas_call(
        _tc1_body,
        grid=(N_NODES // _RB,),
        in_specs=[
            pl.BlockSpec((_RB, NUM_FEATURES), lambda i: (i, 0)),
            pl.BlockSpec((NC, 1, 1, _RB), lambda i: (0, i, 0, 0)),
            pl.BlockSpec((NUM_FEATURES, H), lambda i: (0, 0)),
            pl.BlockSpec((1, H), lambda i: (0, 0)),
            pl.BlockSpec((H, H), lambda i: (0, 0)),
        ],
        out_specs=[
            pl.BlockSpec((_RB, H), lambda i: (i, 0)),
            pl.BlockSpec((1, 1, _RB), lambda i: (i, 0, 0)),
        ],
        out_shape=[
            jax.ShapeDtypeStruct((N_NODES, H), _f32),
            jax.ShapeDtypeStruct((N_NODES // _RB, 1, _RB), _f32),
        ],
    )(feature, deg3, W_enc, b_enc.reshape(1, H), W_gcn)

    agg = _sc_b(y, src1, dst1, zeros2)

    wq2p = jnp.zeros((H, 8), _f32).at[:, :ACTION_DIM].set(W_q2)
    bq2p = jnp.zeros((1, 8), _f32).at[0, :ACTION_DIM].set(b_q2)
    q = pl.pallas_call(
        _tc2_body,
        grid=(N_NODES // _RB,),
        in_specs=[
            pl.BlockSpec((NC, _RB, H), lambda i: (0, i, 0)),
            pl.BlockSpec((_RB, H), lambda i: (i, 0)),
            pl.BlockSpec((1, 1, _RB), lambda i: (i, 0, 0)),
            pl.BlockSpec((1, H), lambda i: (0, 0)),
            pl.BlockSpec((H, H), lambda i: (0, 0)),
            pl.BlockSpec((1, H), lambda i: (0, 0)),
            pl.BlockSpec((H, 8), lambda i: (0, 0)),
            pl.BlockSpec((1, 8), lambda i: (0, 0)),
        ],
        out_specs=pl.BlockSpec((_RB, 8), lambda i: (i, 0)),
        out_shape=jax.ShapeDtypeStruct((N_NODES, 8), _f32),
    )(agg, y, dinv, b_gcn.reshape(1, H), W_q1, b_q1.reshape(1, H),
      wq2p, bq2p)
    return q[:, :ACTION_DIM]


# speedup vs baseline: 42.7364x; 1.1042x over previous
import jax
import jax.numpy as jnp
from jax import lax
from jax.experimental import pallas as pl
from jax.experimental.pallas import tpu as pltpu
from jax.experimental.pallas import tpu_sc as plsc

N_NODES = 10000
NUM_FEATURES = 16
OBS_LEN = N_NODES * NUM_FEATURES
N_EDGES = 320000
H = 128
ACTION_DIM = 4

NC = 2
NS = 16
NW = NC * NS

B = 128
EPW = N_EDGES // NW
FPW = OBS_LEN // NW
N_ACC = 10240
RPS = N_ACC // NS

_EBATCH = [(b * B, B) for b in range(EPW // B)] + [(EPW - EPW % B, EPW % B)]
_FBATCH = [(b * B, B) for b in range(FPW // B)] + [(FPW - FPW % B, FPW % B)]
_GSZ = 2048
_GB = _GSZ // B

_f32 = jnp.float32


def _sc_a_body(obs_hbm, fi_hbm, dst_hbm, zeros1_hbm, ones_hbm,
               feat_out, deg_out,
               fi_v, g_v, dst_v, ones_v, deg_sh, gsem, ssem):
    c = lax.axis_index("c")
    s = lax.axis_index("s")
    w = s * NC + c
    pltpu.sync_copy(zeros1_hbm, deg_sh.at[pl.ds(s * RPS, RPS)])
    pltpu.sync_copy(ones_hbm, ones_v)
    pltpu.sync_copy(fi_hbm.at[pl.ds(w * FPW, FPW)], fi_v)
    gds = [pltpu.async_copy(obs_hbm.at[fi_v.at[pl.ds(off, n)]],
                            g_v.at[pl.ds(off, n)], gsem)
           for off, n in _FBATCH]
    pltpu.sync_copy(dst_hbm.at[pl.ds(w * EPW, EPW)], dst_v)
    for d in gds:
        d.wait()
    pltpu.sync_copy(g_v, feat_out.at[pl.ds(w * FPW, FPW)])
    plsc.subcore_barrier()
    sds = [pltpu.async_copy(ones_v.at[pl.ds(0, n)],
                            deg_sh.at[dst_v.at[pl.ds(off, n)]], ssem,
                            add=True)
           for off, n in _EBATCH]
    for d in sds:
        d.wait()
    plsc.subcore_barrier()
    pltpu.sync_copy(deg_sh.at[pl.ds(s * RPS, RPS)],
                    deg_out.at[c, 0, pl.ds(s * RPS, RPS)])


_sc_a = pl.kernel(
    _sc_a_body,
    out_type=(jax.ShapeDtypeStruct((OBS_LEN,), _f32),
              jax.ShapeDtypeStruct((NC, 1, N_ACC), _f32)),
    mesh=plsc.VectorSubcoreMesh(core_axis_name="c", subcore_axis_name="s",
                                num_cores=NC, num_subcores=NS),
    scratch_types=(
        pltpu.VMEM((FPW,), jnp.int32),
        pltpu.VMEM((FPW,), _f32),
        pltpu.VMEM((EPW,), jnp.int32),
        pltpu.VMEM((B,), _f32),
        pltpu.VMEM_SHARED((N_ACC,), _f32),
        pltpu.SemaphoreType.DMA,
        pltpu.SemaphoreType.DMA,
    ),
)


def _sc_b_body(y_hbm, src_hbm, dst_hbm, zeros2_hbm,
               agg_out,
               src_v0, src_v1, dst_v0, dst_v1, msg_v, agg_sh,
               gsem, ssem, isem):
    src_b = (src_v0, src_v1)
    dst_b = (dst_v0, dst_v1)
    c = lax.axis_index("c")
    s = lax.axis_index("s")
    w = s * NC + c
    for t in range(RPS // B):
        pltpu.sync_copy(zeros2_hbm, agg_sh.at[pl.ds(s * RPS + t * B, B)])

    nb = len(_EBATCH)

    def load_idx(g):
        base = w * EPW + g * _GSZ
        gsz = min(_GSZ, EPW - g * _GSZ)
        return (pltpu.async_copy(src_hbm.at[pl.ds(base, gsz)],
                                 src_b[g % 2].at[pl.ds(0, gsz)], isem),
                pltpu.async_copy(dst_hbm.at[pl.ds(base, gsz)],
                                 dst_b[g % 2].at[pl.ds(0, gsz)], isem))

    def gather(j):
        off, n = _EBATCH[j]
        goff = off - (off // _GSZ) * _GSZ
        return pltpu.async_copy(
            y_hbm.at[src_b[(j // _GB) % 2].at[pl.ds(goff, n)]],
            msg_v.at[j % 2, pl.ds(0, n)], gsem)

    def scatter(j):
        off, n = _EBATCH[j]
        goff = off - (off // _GSZ) * _GSZ
        return pltpu.async_copy(
            msg_v.at[j % 2, pl.ds(0, n)],
            agg_sh.at[dst_b[(j // _GB) % 2].at[pl.ds(goff, n)]],
            ssem, add=True)

    ngrp = (nb + _GB - 1) // _GB
    il = [None] * ngrp
    il[0] = load_idx(0)
    for d in il[0]:
        d.wait()
    plsc.subcore_barrier()

    gd = [None] * nb
    sd = [None] * nb
    gd[0] = gather(0)
    for j in range(nb):
        if j >= 1:
            sd[j - 1].wait()
        if j % _GB == 0 and (g := j // _GB) + 1 < ngrp:
            il[g + 1] = load_idx(g + 1)
        nxt = j + 1
        if nxt < nb:
            if nxt % _GB == 0:
                for d in il[nxt // _GB]:
                    d.wait()
            gd[nxt] = gather(nxt)
        gd[j].wait()
        sd[j] = scatter(j)
    sd[nb - 1].wait()
    plsc.subcore_barrier()
    pltpu.sync_copy(agg_sh.at[pl.ds(s * RPS, RPS)],
                    agg_out.at[c, pl.ds(s * RPS, RPS)])


_sc_b = pl.kernel(
    _sc_b_body,
    out_type=jax.ShapeDtypeStruct((NC, N_ACC, H), _f32),
    mesh=plsc.VectorSubcoreMesh(core_axis_name="c", subcore_axis_name="s",
                                num_cores=NC, num_subcores=NS),
    scratch_types=(
        pltpu.VMEM((_GSZ,), jnp.int32),
        pltpu.VMEM((_GSZ,), jnp.int32),
        pltpu.VMEM((_GSZ,), jnp.int32),
        pltpu.VMEM((_GSZ,), jnp.int32),
        pltpu.VMEM((2, B, H), _f32),
        pltpu.VMEM_SHARED((N_ACC, H), _f32),
        pltpu.SemaphoreType.DMA,
        pltpu.SemaphoreType.DMA,
        pltpu.SemaphoreType.DMA,
    ),
)


_RB = 2000
_EBLK = 32000


def _split_body(ei_ref, src_ref, dst_ref):
    src_ref[...] = ei_ref[0, :]
    dst_ref[...] = ei_ref[1, :]


def _split_edges(edge_index):
    return pl.pallas_call(
        _split_body,
        out_shape=[jax.ShapeDtypeStruct((N_EDGES,), jnp.int32),
                   jax.ShapeDtypeStruct((N_EDGES,), jnp.int32)],
    )(edge_index)


def _tc1_body(feat_ref, deg_ref, we_ref, be_ref, wg_ref, y_ref, dinv_ref):
    f = feat_ref[...]
    h1 = jnp.maximum(
        jnp.dot(f, we_ref[...], preferred_element_type=_f32) + be_ref[...], 0.0)
    xw = jnp.dot(h1, wg_ref[...], preferred_element_type=_f32)
    deg = deg_ref[0, 0, 0, :] + deg_ref[1, 0, 0, :] + 1.0
    dinv = lax.rsqrt(deg)
    y_ref[...] = xw * dinv[:, None]
    dinv_ref[...] = dinv[None, None, :]


def _tc2_body(agg_ref, y_ref, dinv_ref, bg_ref, wq1_ref, bq1_ref,
              wq2_ref, bq2_ref, q_ref):
    pre = (agg_ref[0] + agg_ref[1] + y_ref[...]) * dinv_ref[0, 0, :][:, None]
    h2 = jnp.maximum(pre + bg_ref[...], 0.0)
    t = jnp.maximum(
        jnp.dot(h2, wq1_ref[...], preferred_element_type=_f32) + bq1_ref[...],
        0.0)
    q_ref[...] = jnp.dot(t, wq2_ref[...], preferred_element_type=_f32) \
        + bq2_ref[...]


def kernel(eplus_obs_vec, feature_indices, edge_index, W_enc, b_enc,
           W_gcn, b_gcn, W_q1, b_q1, W_q2, b_q2):
    fi1 = feature_indices.reshape(-1)
    src1, dst1 = _split_edges(edge_index)

    zeros1 = jnp.zeros((RPS,), _f32)
    zeros2 = jnp.zeros((B, H), _f32)
    ones1 = jnp.ones((B,), _f32)

    feat_flat, deg_parts = _sc_a(eplus_obs_vec, fi1, dst1, zeros1, ones1)
    feature = feat_flat.reshape(N_NODES, NUM_FEATURES)
    deg3 = deg_parts[:, 0, :N_NODES].reshape(NC, N_NODES // _RB, 1, _RB)

    y, dinv = pl.pallas_call(
        _tc1_body,
        grid=(N_NODES // _RB,),
        in_specs=[
            pl.BlockSpec((_RB, NUM_FEATURES), lambda i: (i, 0)),
            pl.BlockSpec((NC, 1, 1, _RB), lambda i: (0, i, 0, 0)),
            pl.BlockSpec((NUM_FEATURES, H), lambda i: (0, 0)),
            pl.BlockSpec((1, H), lambda i: (0, 0)),
            pl.BlockSpec((H, H), lambda i: (0, 0)),
        ],
        out_specs=[
            pl.BlockSpec((_RB, H), lambda i: (i, 0)),
            pl.BlockSpec((1, 1, _RB), lambda i: (i, 0, 0)),
        ],
        out_shape=[
            jax.ShapeDtypeStruct((N_NODES, H), _f32),
            jax.ShapeDtypeStruct((N_NODES // _RB, 1, _RB), _f32),
        ],
    )(feature, deg3, W_enc, b_enc.reshape(1, H), W_gcn)

    agg = _sc_b(y, src1, dst1, zeros2)

    wq2p = W_q2
    bq2p = b_q2.reshape(1, ACTION_DIM)
    q = pl.pallas_call(
        _tc2_body,
        grid=(N_NODES // _RB,),
        in_specs=[
            pl.BlockSpec((NC, _RB, H), lambda i: (0, i, 0)),
            pl.BlockSpec((_RB, H), lambda i: (i, 0)),
            pl.BlockSpec((1, 1, _RB), lambda i: (i, 0, 0)),
            pl.BlockSpec((1, H), lambda i: (0, 0)),
            pl.BlockSpec((H, H), lambda i: (0, 0)),
            pl.BlockSpec((1, H), lambda i: (0, 0)),
            pl.BlockSpec((H, ACTION_DIM), lambda i: (0, 0)),
            pl.BlockSpec((1, ACTION_DIM), lambda i: (0, 0)),
        ],
        out_specs=pl.BlockSpec((_RB, ACTION_DIM), lambda i: (i, 0)),
        out_shape=jax.ShapeDtypeStruct((N_NODES, ACTION_DIM), _f32),
    )(agg, y, dinv, b_gcn.reshape(1, H), W_q1, b_q1.reshape(1, H),
      wq2p, bq2p)
    return q


# speedup vs baseline: 42.8042x; 1.0016x over previous
import functools

import jax
import jax.numpy as jnp
from jax import lax
from jax.experimental import pallas as pl
from jax.experimental.pallas import tpu as pltpu
from jax.experimental.pallas import tpu_sc as plsc

N_NODES = 10000
NUM_FEATURES = 16
OBS_LEN = N_NODES * NUM_FEATURES
N_EDGES = 320000
H = 128
ACTION_DIM = 4

NC = 2
NS = 16
NW = NC * NS

B = 128
EPW = N_EDGES // NW
FPW = OBS_LEN // NW
N_ACC = 10240
RPS = N_ACC // NS

_EBATCH = [(b * B, B) for b in range(EPW // B)] + [(EPW - EPW % B, EPW % B)]
_FBATCH = [(b * B, B) for b in range(FPW // B)] + [(FPW - FPW % B, FPW % B)]
_GSZ = 2048
_GB = _GSZ // B

_f32 = jnp.float32


def _sc_a_body(obs_hbm, fi_hbm, dst_hbm, zeros1_hbm, ones_hbm,
               feat_out, deg_out,
               fi_v, g_v, dst_v, ones_v, deg_sh, gsem, ssem):
    c = lax.axis_index("c")
    s = lax.axis_index("s")
    w = s * NC + c
    pltpu.sync_copy(zeros1_hbm, deg_sh.at[pl.ds(s * RPS, RPS)])
    pltpu.sync_copy(ones_hbm, ones_v)
    pltpu.sync_copy(fi_hbm.at[pl.ds(w * FPW, FPW)], fi_v)
    gds = [pltpu.async_copy(obs_hbm.at[fi_v.at[pl.ds(off, n)]],
                            g_v.at[pl.ds(off, n)], gsem)
           for off, n in _FBATCH]
    pltpu.sync_copy(dst_hbm.at[pl.ds(w * EPW, EPW)], dst_v)
    for d in gds:
        d.wait()
    pltpu.sync_copy(g_v, feat_out.at[pl.ds(w * FPW, FPW)])
    plsc.subcore_barrier()
    sds = [pltpu.async_copy(ones_v.at[pl.ds(0, n)],
                            deg_sh.at[dst_v.at[pl.ds(off, n)]], ssem,
                            add=True)
           for off, n in _EBATCH]
    for d in sds:
        d.wait()
    plsc.subcore_barrier()
    pltpu.sync_copy(deg_sh.at[pl.ds(s * RPS, RPS)],
                    deg_out.at[c, 0, pl.ds(s * RPS, RPS)])


@functools.lru_cache(maxsize=None)
def _build_sc_a():
  return pl.kernel(
    _sc_a_body,
    out_type=(jax.ShapeDtypeStruct((OBS_LEN,), _f32),
              jax.ShapeDtypeStruct((NC, 1, N_ACC), _f32)),
    mesh=plsc.VectorSubcoreMesh(core_axis_name="c", subcore_axis_name="s",
                                num_cores=NC, num_subcores=NS),
    scratch_types=(
        pltpu.VMEM((FPW,), jnp.int32),
        pltpu.VMEM((FPW,), _f32),
        pltpu.VMEM((EPW,), jnp.int32),
        pltpu.VMEM((B,), _f32),
        pltpu.VMEM_SHARED((N_ACC,), _f32),
        pltpu.SemaphoreType.DMA,
        pltpu.SemaphoreType.DMA,
    ),
  )


def _sc_b_body(y_hbm, src_hbm, dst_hbm, zeros2_hbm,
               agg_out,
               src_v0, src_v1, dst_v0, dst_v1, msg_v, agg_sh,
               gsem, ssem, isem):
    src_b = (src_v0, src_v1)
    dst_b = (dst_v0, dst_v1)
    c = lax.axis_index("c")
    s = lax.axis_index("s")
    w = s * NC + c
    for t in range(RPS // B):
        pltpu.sync_copy(zeros2_hbm, agg_sh.at[pl.ds(s * RPS + t * B, B)])

    nb = len(_EBATCH)

    def load_idx(g):
        base = w * EPW + g * _GSZ
        gsz = min(_GSZ, EPW - g * _GSZ)
        return (pltpu.async_copy(src_hbm.at[pl.ds(base, gsz)],
                                 src_b[g % 2].at[pl.ds(0, gsz)], isem),
                pltpu.async_copy(dst_hbm.at[pl.ds(base, gsz)],
                                 dst_b[g % 2].at[pl.ds(0, gsz)], isem))

    def gather(j):
        off, n = _EBATCH[j]
        goff = off - (off // _GSZ) * _GSZ
        return pltpu.async_copy(
            y_hbm.at[src_b[(j // _GB) % 2].at[pl.ds(goff, n)]],
            msg_v.at[j % 2, pl.ds(0, n)], gsem)

    def scatter(j):
        off, n = _EBATCH[j]
        goff = off - (off // _GSZ) * _GSZ
        return pltpu.async_copy(
            msg_v.at[j % 2, pl.ds(0, n)],
            agg_sh.at[dst_b[(j // _GB) % 2].at[pl.ds(goff, n)]],
            ssem, add=True)

    ngrp = (nb + _GB - 1) // _GB
    il = [None] * ngrp
    il[0] = load_idx(0)
    for d in il[0]:
        d.wait()
    plsc.subcore_barrier()

    gd = [None] * nb
    sd = [None] * nb
    gd[0] = gather(0)
    for j in range(nb):
        if j >= 1:
            sd[j - 1].wait()
        if j % _GB == 0 and (g := j // _GB) + 1 < ngrp:
            il[g + 1] = load_idx(g + 1)
        nxt = j + 1
        if nxt < nb:
            if nxt % _GB == 0:
                for d in il[nxt // _GB]:
                    d.wait()
            gd[nxt] = gather(nxt)
        gd[j].wait()
        sd[j] = scatter(j)
    sd[nb - 1].wait()
    plsc.subcore_barrier()
    pltpu.sync_copy(agg_sh.at[pl.ds(s * RPS, RPS)],
                    agg_out.at[c, pl.ds(s * RPS, RPS)])


@functools.lru_cache(maxsize=None)
def _build_sc_b():
  return pl.kernel(
    _sc_b_body,
    out_type=jax.ShapeDtypeStruct((NC, N_ACC, H), _f32),
    mesh=plsc.VectorSubcoreMesh(core_axis_name="c", subcore_axis_name="s",
                                num_cores=NC, num_subcores=NS),
    scratch_types=(
        pltpu.VMEM((_GSZ,), jnp.int32),
        pltpu.VMEM((_GSZ,), jnp.int32),
        pltpu.VMEM((_GSZ,), jnp.int32),
        pltpu.VMEM((_GSZ,), jnp.int32),
        pltpu.VMEM((2, B, H), _f32),
        pltpu.VMEM_SHARED((N_ACC, H), _f32),
        pltpu.SemaphoreType.DMA,
        pltpu.SemaphoreType.DMA,
        pltpu.SemaphoreType.DMA,
    ),
  )


_RB = 2000
_EBLK = 32000


def _split_body(ei_ref, src_ref, dst_ref):
    src_ref[...] = ei_ref[0, :]
    dst_ref[...] = ei_ref[1, :]


def _split_edges(edge_index):
    return pl.pallas_call(
        _split_body,
        out_shape=[jax.ShapeDtypeStruct((N_EDGES,), jnp.int32),
                   jax.ShapeDtypeStruct((N_EDGES,), jnp.int32)],
    )(edge_index)


def _tc1_body(feat_ref, deg_ref, we_ref, be_ref, wg_ref, y_ref, dinv_ref):
    f = feat_ref[...]
    h1 = jnp.maximum(
        jnp.dot(f, we_ref[...], preferred_element_type=_f32) + be_ref[...], 0.0)
    xw = jnp.dot(h1, wg_ref[...], preferred_element_type=_f32)
    deg = deg_ref[0, 0, 0, :] + deg_ref[1, 0, 0, :] + 1.0
    dinv = lax.rsqrt(deg)
    y_ref[...] = xw * dinv[:, None]
    dinv_ref[...] = dinv[None, None, :]


def _tc2_body(agg_ref, y_ref, dinv_ref, bg_ref, wq1_ref, bq1_ref,
              wq2_ref, bq2_ref, q_ref):
    pre = (agg_ref[0] + agg_ref[1] + y_ref[...]) * dinv_ref[0, 0, :][:, None]
    h2 = jnp.maximum(pre + bg_ref[...], 0.0)
    t = jnp.maximum(
        jnp.dot(h2, wq1_ref[...], preferred_element_type=_f32) + bq1_ref[...],
        0.0)
    q_ref[...] = jnp.dot(t, wq2_ref[...], preferred_element_type=_f32) \
        + bq2_ref[...]


def kernel(eplus_obs_vec, feature_indices, edge_index, W_enc, b_enc,
           W_gcn, b_gcn, W_q1, b_q1, W_q2, b_q2):
    fi1 = feature_indices.reshape(-1)
    src1, dst1 = _split_edges(edge_index)

    zeros1 = jnp.zeros((RPS,), _f32)
    zeros2 = jnp.zeros((B, H), _f32)
    ones1 = jnp.ones((B,), _f32)

    feat_flat, deg_parts = _build_sc_a()(eplus_obs_vec, fi1, dst1, zeros1,
                                        ones1)
    feature = feat_flat.reshape(N_NODES, NUM_FEATURES)
    deg3 = deg_parts[:, 0, :N_NODES].reshape(NC, N_NODES // _RB, 1, _RB)

    y, dinv = pl.pallas_call(
        _tc1_body,
        grid=(N_NODES // _RB,),
        in_specs=[
            pl.BlockSpec((_RB, NUM_FEATURES), lambda i: (i, 0)),
            pl.BlockSpec((NC, 1, 1, _RB), lambda i: (0, i, 0, 0)),
            pl.BlockSpec((NUM_FEATURES, H), lambda i: (0, 0)),
            pl.BlockSpec((1, H), lambda i: (0, 0)),
            pl.BlockSpec((H, H), lambda i: (0, 0)),
        ],
        out_specs=[
            pl.BlockSpec((_RB, H), lambda i: (i, 0)),
            pl.BlockSpec((1, 1, _RB), lambda i: (i, 0, 0)),
        ],
        out_shape=[
            jax.ShapeDtypeStruct((N_NODES, H), _f32),
            jax.ShapeDtypeStruct((N_NODES // _RB, 1, _RB), _f32),
        ],
    )(feature, deg3, W_enc, b_enc.reshape(1, H), W_gcn)

    agg = _build_sc_b()(y, src1, dst1, zeros2)

    wq2p = W_q2
    bq2p = b_q2.reshape(1, ACTION_DIM)
    q = pl.pallas_call(
        _tc2_body,
        grid=(N_NODES // _RB,),
        in_specs=[
            pl.BlockSpec((NC, _RB, H), lambda i: (0, i, 0)),
            pl.BlockSpec((_RB, H), lambda i: (i, 0)),
            pl.BlockSpec((1, 1, _RB), lambda i: (i, 0, 0)),
            pl.BlockSpec((1, H), lambda i: (0, 0)),
            pl.BlockSpec((H, H), lambda i: (0, 0)),
            pl.BlockSpec((1, H), lambda i: (0, 0)),
            pl.BlockSpec((H, ACTION_DIM), lambda i: (0, 0)),
            pl.BlockSpec((1, ACTION_DIM), lambda i: (0, 0)),
        ],
        out_specs=pl.BlockSpec((_RB, ACTION_DIM), lambda i: (i, 0)),
        out_shape=jax.ShapeDtypeStruct((N_NODES, ACTION_DIM), _f32),
    )(agg, y, dinv, b_gcn.reshape(1, H), W_q1, b_q1.reshape(1, H),
      wq2p, bq2p)
    return q
